# Initial kernel scaffold; baseline (speedup 1.0000x reference)
import jax
import jax.numpy as jnp
from jax.experimental import pallas as pl


def kernel(h, e, edge_index, A_w, A_b, B_w, B_b, C_w, C_b, Dm_w, Dm_b, U_w, U_b, V_w, V_b, bn_g, bn_b):
    raise NotImplementedError("write your pallas kernel here")


# speedup vs baseline: 2.0576x; 2.0576x over previous
import functools

import jax
import jax.numpy as jnp
from jax import lax
from jax.experimental import pallas as pl
from jax.experimental.pallas import tpu as pltpu
from jax.experimental.pallas import tpu_sc as plsc

EPS = 1e-05
BN_EPS = 1e-05

N = 100000
D = 128
E = 100000

NC = 2
NS = 16
NW = NC * NS
CHUNK = 3200
NP = NW * CHUNK
GW = 128
NWIN = CHUNK // GW
EW = 4000
PADCNT = NP_PAD_STATS = 96
EPAD = 100352
DUMMY_SPREAD = 256

_mesh = plsc.VectorSubcoreMesh(core_axis_name="c", subcore_axis_name="s")
f32 = jnp.float32
i32 = jnp.int32


def _wid():
    return lax.axis_index("s") * NC + lax.axis_index("c")


@functools.partial(
    pl.kernel,
    out_type=jax.ShapeDtypeStruct((NP,), i32),
    mesh=_mesh,
    scratch_types=[
        pltpu.VMEM((CHUNK,), i32),
        pltpu.VMEM((EW,), i32),
    ],
    compiler_params=pltpu.CompilerParams(needs_layout_passes=False),
)
def _winner_sc(row_hbm, wd_hbm, wloc, rbuf):
    wid = _wid()
    n0 = wid * CHUNK
    iota = lax.iota(i32, 16)

    def init_body(i, _):
        wloc[pl.ds(i * 16, 16)] = jnp.full((16,), -1, i32)
        return 0

    lax.fori_loop(0, CHUNK // 16, init_body, 0)

    def win_body(w, _):
        eb = w * EW
        pltpu.sync_copy(row_hbm.at[pl.ds(eb, EW)], rbuf)

        def body(k, _2):
            v = rbuf[pl.ds(k * 16, 16)]
            local = v - n0
            m = (local >= 0) & (local < CHUNK)
            lc = jnp.clip(local, 0, CHUNK - 1)
            jv = (eb + k * 16) + iota
            plsc.store_scatter(wloc, [lc], jv, mask=m)
            return 0

        lax.fori_loop(0, EW // 16, body, 0)
        return 0

    lax.fori_loop(0, E // EW, win_body, 0)

    def fin_body(i, _):
        sl = pl.ds(i * 16, 16)
        wv = wloc[sl]
        n = (n0 + i * 16) + iota
        wloc[sl] = jnp.where(wv < 0, E + (n & (DUMMY_SPREAD - 1)), wv)
        return 0

    lax.fori_loop(0, CHUNK // 16, fin_body, 0)
    pltpu.sync_copy(wloc, wd_hbm.at[pl.ds(n0, CHUNK)])


@functools.partial(
    pl.kernel,
    out_type=(
        jax.ShapeDtypeStruct((NP, D), f32),
        jax.ShapeDtypeStruct((NP, D), f32),
        jax.ShapeDtypeStruct((NW, 2 * D), f32),
    ),
    mesh=_mesh,
    scratch_types=[
        pltpu.VMEM((GW,), i32),
        pltpu.VMEM((GW,), i32),
        pltpu.VMEM((GW, D), f32),
        pltpu.VMEM((GW, D), f32),
        pltpu.VMEM((GW, D), f32),
        pltpu.VMEM((2 * D,), f32),
        pltpu.SemaphoreType.DMA,
    ],
)
def _edge_gather_sc(rowp_hbm, colp_hbm, p_hbm, q_hbm, v_hbm,
                    g12_hbm, g3_hbm, stats_hbm,
                    ridx, cidx, bufp, bufq, bufv, sacc, sem):
    wid = _wid()
    base = wid * CHUNK

    for c in range(2 * D // 16):
        sacc[pl.ds(c * 16, 16)] = jnp.zeros((16,), f32)

    def win_body(w, _):
        wb = base + w * GW

        @pl.when(wb < E)
        def _process():
            pltpu.sync_copy(rowp_hbm.at[pl.ds(wb, GW)], ridx)
            pltpu.sync_copy(colp_hbm.at[pl.ds(wb, GW)], cidx)
            cp1 = pltpu.async_copy(p_hbm.at[ridx], bufp, sem)
            cp2 = pltpu.async_copy(q_hbm.at[cidx], bufq, sem)
            cp3 = pltpu.async_copy(v_hbm.at[cidx], bufv, sem)
            cp1.wait()
            cp2.wait()
            cp3.wait()

            def row_body(r, acc):
                out = []
                for c in range(D // 16):
                    sl = pl.ds(c * 16, 16)
                    g = bufp[r, sl] + bufq[r, sl]
                    bufp[r, sl] = g
                    out.append(acc[2 * c] + g)
                    out.append(acc[2 * c + 1] + g * g)
                return tuple(out)

            acc = lax.fori_loop(
                0, GW, row_body,
                tuple(jnp.zeros((16,), f32) for _ in range(2 * (D // 16))))
            for c in range(D // 16):
                s1 = pl.ds(c * 16, 16)
                s2 = pl.ds(D + c * 16, 16)
                sacc[s1] = sacc[s1] + acc[2 * c]
                sacc[s2] = sacc[s2] + acc[2 * c + 1]
            pltpu.sync_copy(bufp, g12_hbm.at[pl.ds(wb, GW)])
            pltpu.sync_copy(bufv, g3_hbm.at[pl.ds(wb, GW)])

        return 0

    lax.fori_loop(0, NWIN, win_body, 0)
    pltpu.sync_copy(sacc, stats_hbm.at[wid])


@functools.partial(
    pl.kernel,
    out_type=jax.ShapeDtypeStruct((NP, D), f32),
    mesh=_mesh,
    scratch_types=[
        pltpu.VMEM((GW,), i32),
        pltpu.VMEM((GW, D), f32),
        pltpu.SemaphoreType.DMA,
    ],
)
def _node_gather_sc(wd_hbm, eout_hbm, c_hbm, idxb, bufc, sem):
    wid = _wid()
    base = wid * CHUNK

    def win_body(w, _):
        nb = base + w * GW

        @pl.when(nb < E)
        def _process():
            pltpu.sync_copy(wd_hbm.at[pl.ds(nb, GW)], idxb)
            pltpu.async_copy(eout_hbm.at[idxb], bufc, sem).wait()
            pltpu.sync_copy(bufc, c_hbm.at[pl.ds(nb, GW)])

        return 0

    lax.fori_loop(0, NWIN, win_body, 0)


_DN = (((1,), (1,)), ((), ()))


def _mm_body(h_r, e_r, wa, wd, wb, wc, wv, bp, bq, bv, p_r, q_r, v_r):
    hb = h_r[...]
    eb = e_r[...]
    p_r[...] = (lax.dot_general(hb, wa[...], _DN, preferred_element_type=f32)
                + lax.dot_general(eb, wd[...], _DN, preferred_element_type=f32)
                + bp[...])
    q_r[...] = (lax.dot_general(hb, wb[...], _DN, preferred_element_type=f32)
                + lax.dot_general(eb, wc[...], _DN, preferred_element_type=f32)
                + bq[...])
    v_r[...] = lax.dot_general(hb, wv[...], _DN, preferred_element_type=f32) + bv[...]


def _bn_body(g12_r, stats_r, gpad_r, bng_r, bnb_r, out_r):
    stats = stats_r[...]
    gpad = gpad_r[...]
    ssum = jnp.sum(stats[:, :D], axis=0, keepdims=True) - PADCNT * gpad
    ssq = jnp.sum(stats[:, D:], axis=0, keepdims=True) - PADCNT * gpad * gpad
    mean = ssum / E
    var = ssq / E - mean * mean
    a = lax.rsqrt(var + BN_EPS) * bng_r[...]
    b = pl.program_id(0)
    rows = b * 512 + lax.broadcasted_iota(i32, (512, D), 0)
    val = jax.nn.relu((g12_r[...] - mean) * a + bnb_r[...])
    out_r[...] = jnp.where(rows < E, val, 0.0)


def _sig_body(e_r, c_r, g3_r, s_r, acc_r):
    @pl.when(pl.program_id(0) == 0)
    def _init():
        acc_r[...] = jnp.zeros((2, D), f32)

    s = jax.nn.sigmoid(e_r[...] + c_r[...])
    s_r[...] = s
    p0 = jnp.sum(s, axis=0)
    p1 = jnp.sum(s * g3_r[...], axis=0)
    acc_r[...] = acc_r[...] + jnp.stack([p0, p1], axis=0)


def _final_body(h_r, s_r, acc_r, uw, ub, hout_r, enew_r):
    inv = 1.0 / (acc_r[0:1, :] + EPS)
    enew_r[...] = s_r[...] * inv
    hout_r[...] = jax.nn.relu(
        lax.dot_general(h_r[...], uw[...], _DN, preferred_element_type=f32)
        + ub[...] + acc_r[1:2, :] * inv)


def _full(shape):
    nd = len(shape)
    return pl.BlockSpec(shape, lambda b: (0,) * nd)


def kernel(h, e, edge_index, A_w, A_b, B_w, B_b, C_w, C_b, Dm_w, Dm_b,
           U_w, U_b, V_w, V_b, bn_g, bn_b):
    row = edge_index[0]
    col = edge_index[1]
    pad = jnp.zeros((NP - E,), i32)
    rowp = jnp.concatenate([row, pad])
    colp = jnp.concatenate([col, pad])
    bp = (A_b + Dm_b).reshape(1, D)
    bq = (B_b + C_b).reshape(1, D)
    bv = V_b.reshape(1, D)

    wd = _winner_sc(row)

    blk = 1000
    g1 = N // blk
    P, Q, V = pl.pallas_call(
        _mm_body,
        grid=(g1,),
        in_specs=[pl.BlockSpec((blk, D), lambda b: (b, 0)),
                  pl.BlockSpec((blk, D), lambda b: (b, 0)),
                  _full((D, D)), _full((D, D)), _full((D, D)),
                  _full((D, D)), _full((D, D)),
                  _full((1, D)), _full((1, D)), _full((1, D))],
        out_specs=[pl.BlockSpec((blk, D), lambda b: (b, 0))] * 3,
        out_shape=[jax.ShapeDtypeStruct((N, D), f32)] * 3,
    )(h, e, A_w, Dm_w, B_w, C_w, V_w, bp, bq, bv)

    G12, G3, stats = _edge_gather_sc(rowp, colp, P, Q, V)

    gpad = lax.slice(G12, (E, 0), (E + 1, D))
    e_out = pl.pallas_call(
        _bn_body,
        grid=(EPAD // 512,),
        in_specs=[pl.BlockSpec((512, D), lambda b: (b, 0)),
                  _full((NW, 2 * D)), _full((1, D)),
                  _full((1, D)), _full((1, D))],
        out_specs=pl.BlockSpec((512, D), lambda b: (b, 0)),
        out_shape=jax.ShapeDtypeStruct((EPAD, D), f32),
    )(G12, stats, gpad, bn_g.reshape(1, D), bn_b.reshape(1, D))

    c = _node_gather_sc(wd, e_out)

    blk2 = 800
    g2 = N // blk2
    s, acc = pl.pallas_call(
        _sig_body,
        grid=(g2,),
        in_specs=[pl.BlockSpec((blk2, D), lambda b: (b, 0)),
                  pl.BlockSpec((blk2, D), lambda b: (b, 0)),
                  pl.BlockSpec((blk2, D), lambda b: (b, 0))],
        out_specs=[pl.BlockSpec((blk2, D), lambda b: (b, 0)),
                   _full((2, D))],
        out_shape=[jax.ShapeDtypeStruct((N, D), f32),
                   jax.ShapeDtypeStruct((2, D), f32)],
    )(e, c, G3)

    h_out, e_new = pl.pallas_call(
        _final_body,
        grid=(g2,),
        in_specs=[pl.BlockSpec((blk2, D), lambda b: (b, 0)),
                  pl.BlockSpec((blk2, D), lambda b: (b, 0)),
                  _full((2, D)), _full((D, D)), _full((1, D))],
        out_specs=[pl.BlockSpec((blk2, D), lambda b: (b, 0))] * 2,
        out_shape=[jax.ShapeDtypeStruct((N, D), f32)] * 2,
    )(h, s, acc, U_w, U_b.reshape(1, D))

    return (h_out, e_new)
